# speedup vs baseline: 8.9276x; 8.9276x over previous
import functools

import jax
import jax.numpy as jnp
from jax import lax
from jax.experimental import pallas as pl
from jax.experimental.pallas import tpu as pltpu
from jax.experimental.pallas import tpu_sc as plsc

N = 10000
E = 160000
D = 256
H = 128
NC = 2
NS = 16
CH = 128
EPAD = 163840
NROWS = 10240
DUMMY = 10000
RPT = NROWS // NS
CHA = EPAD // (NC * NS * CH)
CHC = EPAD // (NS * CH)


def _sc_mesh():
    return plsc.VectorSubcoreMesh(
        core_axis_name="c", subcore_axis_name="s", num_cores=NC,
        num_subcores=NS)


def _degree(col3a):
    @functools.partial(
        pl.kernel,
        out_type=jax.ShapeDtypeStruct((NC, NROWS, 16), jnp.float32),
        mesh=_sc_mesh(),
        scratch_types=[
            pltpu.VMEM((CHA, CH), jnp.int32),
            pltpu.VMEM((CH, 16), jnp.float32),
            pltpu.VMEM((64, 16), jnp.float32),
            pltpu.VMEM_SHARED((NROWS, 16), jnp.float32),
        ],
    )
    def k(col_hbm, out_hbm, idx_v, ones_v, z_v, acc_sh):
        c = lax.axis_index("c")
        s = lax.axis_index("s")
        wid = s * NC + c

        @pl.loop(0, 64)
        def _(i):
            z_v[i, :] = jnp.zeros((16,), jnp.float32)

        @pl.loop(0, CH)
        def _(i):
            ones_v[i, :] = jnp.ones((16,), jnp.float32)

        @pl.loop(0, RPT // 64)
        def _(kk):
            pltpu.sync_copy(z_v, acc_sh.at[pl.ds(s * RPT + kk * 64, 64)])

        pltpu.sync_copy(col_hbm.at[wid], idx_v)
        plsc.subcore_barrier()

        @pl.loop(0, CHA)
        def _(j):
            pltpu.sync_copy(ones_v, acc_sh.at[idx_v.at[j]], add=True)

        plsc.subcore_barrier()
        pltpu.sync_copy(acc_sh.at[pl.ds(s * RPT, RPT)],
                        out_hbm.at[c, pl.ds(s * RPT, RPT)])

    return k(col3a)


def _segment_sum(lo, hi, row3, col3):
    @functools.partial(
        pl.kernel,
        out_type=jax.ShapeDtypeStruct((NC, NROWS, H), jnp.float32),
        mesh=_sc_mesh(),
        scratch_types=[
            pltpu.VMEM((CHC, CH), jnp.int32),
            pltpu.VMEM((CHC, CH), jnp.int32),
            pltpu.VMEM((CH, H), jnp.float32),
            pltpu.VMEM((64, H), jnp.float32),
            pltpu.VMEM_SHARED((NROWS, H), jnp.float32),
        ],
    )
    def k(lo_hbm, hi_hbm, row_hbm, col_hbm, out_hbm,
          ri_v, ci_v, g_v, z_v, acc_sh):
        c = lax.axis_index("c")
        s = lax.axis_index("s")

        @pl.loop(0, 64)
        def _(i):
            @pl.loop(0, H // 16)
            def _(j):
                z_v[i, pl.ds(j * 16, 16)] = jnp.zeros((16,), jnp.float32)

        @pl.loop(0, RPT // 64)
        def _(kk):
            pltpu.sync_copy(z_v, acc_sh.at[pl.ds(s * RPT + kk * 64, 64)])

        pltpu.sync_copy(row_hbm.at[s], ri_v)
        pltpu.sync_copy(col_hbm.at[s], ci_v)
        plsc.subcore_barrier()

        @pl.loop(0, CHC)
        def _(j):
            @pl.when(c == 0)
            def _():
                pltpu.sync_copy(lo_hbm.at[ri_v.at[j]], g_v)

            @pl.when(c == 1)
            def _():
                pltpu.sync_copy(hi_hbm.at[ri_v.at[j]], g_v)

            pltpu.sync_copy(g_v, acc_sh.at[ci_v.at[j]], add=True)

        plsc.subcore_barrier()
        pltpu.sync_copy(acc_sh.at[pl.ds(s * RPT, RPT)],
                        out_hbm.at[c, pl.ds(s * RPT, RPT)])

    return k(lo, hi, row3, col3)


def _scale_body(x_ref, w_ref, d_ref, lo_ref, hi_ref):
    xw = jnp.dot(x_ref[...], w_ref[...], preferred_element_type=jnp.float32)
    deg = d_ref[0, :, 0] + d_ref[1, :, 0] + 1.0
    dinv = lax.rsqrt(deg)[:, None]
    lo_ref[...] = xw[:, :H] * dinv
    hi_ref[...] = xw[:, H:] * dinv


def _scale(x, w, deg2):
    nb = 10
    bs = N // nb
    return pl.pallas_call(
        _scale_body,
        grid=(nb,),
        in_specs=[
            pl.BlockSpec((bs, D), lambda i: (i, 0)),
            pl.BlockSpec((D, D), lambda i: (0, 0)),
            pl.BlockSpec((NC, bs, 16), lambda i: (0, i, 0)),
        ],
        out_specs=[
            pl.BlockSpec((bs, H), lambda i: (i, 0)),
            pl.BlockSpec((bs, H), lambda i: (i, 0)),
        ],
        out_shape=[
            jax.ShapeDtypeStruct((N, H), jnp.float32),
            jax.ShapeDtypeStruct((N, H), jnp.float32),
        ],
    )(x, w, deg2)


def _fuse_body(hraw_ref, lo_ref, hi_ref, d_ref, bg_ref, t_ref, wt_ref,
               bt_ref, h_ref, st_ref, temb_s, acc_s):
    i = pl.program_id(0)

    @pl.when(i == 0)
    def _():
        temb_s[...] = jnp.maximum(
            jnp.dot(t_ref[...], wt_ref[...],
                    preferred_element_type=jnp.float32) + bt_ref[...], 0.0)
        acc_s[...] = jnp.zeros_like(acc_s)

    temb = temb_s[...]
    deg = d_ref[0, :, 0] + d_ref[1, :, 0] + 1.0
    dinv = lax.rsqrt(deg)[:, None]
    h_lo = jnp.maximum(
        dinv * (hraw_ref[0] + lo_ref[...]) + bg_ref[:, :H] + temb[:, :H], 0.0)
    h_hi = jnp.maximum(
        dinv * (hraw_ref[1] + hi_ref[...]) + bg_ref[:, H:] + temb[:, H:], 0.0)
    h = jnp.concatenate([h_lo, h_hi], axis=1)
    h_ref[...] = h
    acc_s[0:1, :] += jnp.sum(h, axis=0, keepdims=True)
    acc_s[1:2, :] += jnp.sum(h * h, axis=0, keepdims=True)
    st_ref[...] = acc_s[...]


def _fuse(hraw, lo, hi, deg2, bg, t, wt, bt):
    nb = 10
    bs = N // nb
    return pl.pallas_call(
        _fuse_body,
        grid=(nb,),
        in_specs=[
            pl.BlockSpec((NC, bs, H), lambda i: (0, i, 0)),
            pl.BlockSpec((bs, H), lambda i: (i, 0)),
            pl.BlockSpec((bs, H), lambda i: (i, 0)),
            pl.BlockSpec((NC, bs, 16), lambda i: (0, i, 0)),
            pl.BlockSpec((1, D), lambda i: (0, 0)),
            pl.BlockSpec((1, D), lambda i: (0, 0)),
            pl.BlockSpec((D, D), lambda i: (0, 0)),
            pl.BlockSpec((1, D), lambda i: (0, 0)),
        ],
        out_specs=[
            pl.BlockSpec((bs, D), lambda i: (i, 0)),
            pl.BlockSpec((8, D), lambda i: (0, 0)),
        ],
        out_shape=[
            jax.ShapeDtypeStruct((N, D), jnp.float32),
            jax.ShapeDtypeStruct((8, D), jnp.float32),
        ],
        scratch_shapes=[
            pltpu.VMEM((1, D), jnp.float32),
            pltpu.VMEM((8, D), jnp.float32),
        ],
    )(hraw, lo, hi, deg2, bg, t, wt, bt)


def _bn_body(h_ref, st_ref, g_ref, b_ref, o_ref):
    mean = st_ref[0:1, :] * (1.0 / N)
    var = st_ref[1:2, :] * (1.0 / N) - mean * mean
    scale = lax.rsqrt(var + 1e-5) * g_ref[...]
    o_ref[...] = (h_ref[...] - mean) * scale + b_ref[...]


def _bn(h, st, g, b):
    nb = 10
    bs = N // nb
    return pl.pallas_call(
        _bn_body,
        grid=(nb,),
        in_specs=[
            pl.BlockSpec((bs, D), lambda i: (i, 0)),
            pl.BlockSpec((8, D), lambda i: (0, 0)),
            pl.BlockSpec((1, D), lambda i: (0, 0)),
            pl.BlockSpec((1, D), lambda i: (0, 0)),
        ],
        out_specs=pl.BlockSpec((bs, D), lambda i: (i, 0)),
        out_shape=jax.ShapeDtypeStruct((N, D), jnp.float32),
    )(h, st, g, b)


def kernel(x, edge_index, t, W_gcn, b_gcn, W_t, b_t, gamma, beta):
    row = edge_index[0]
    col = edge_index[1]
    pad = EPAD - E
    rowp = jnp.concatenate([row, jnp.zeros((pad,), jnp.int32)])
    colp = jnp.concatenate([col, jnp.full((pad,), DUMMY, jnp.int32)])
    col3a = colp.reshape(NC * NS, CHA, CH)
    row3c = rowp.reshape(NS, CHC, CH)
    col3c = colp.reshape(NS, CHC, CH)

    deg2 = _degree(col3a)
    lo, hi = _scale(x, W_gcn, deg2)
    hraw = _segment_sum(lo, hi, row3c, col3c)
    h, st = _fuse(hraw, lo, hi, deg2, b_gcn.reshape(1, D), t, W_t,
                  b_t.reshape(1, D))
    return _bn(h, st, gamma.reshape(1, D), beta.reshape(1, D))


# speedup vs baseline: 9.8458x; 1.1028x over previous
import functools

import jax
import jax.numpy as jnp
from jax import lax
from jax.experimental import pallas as pl
from jax.experimental.pallas import tpu as pltpu
from jax.experimental.pallas import tpu_sc as plsc

N = 10000
E = 160000
D = 256
H = 128
NC = 2
NS = 16
CH = 128
EPAD = 163840
NROWS = 10240
DUMMY = 10000
RPT = NROWS // NS
CHA = EPAD // (NC * NS * CH)
CHC = EPAD // (NS * CH)


def _sc_mesh():
    return plsc.VectorSubcoreMesh(
        core_axis_name="c", subcore_axis_name="s", num_cores=NC,
        num_subcores=NS)


def _degree(col3a):
    @functools.partial(
        pl.kernel,
        out_type=jax.ShapeDtypeStruct((NC, NROWS, 16), jnp.float32),
        mesh=_sc_mesh(),
        scratch_types=[
            pltpu.VMEM((CHA, CH), jnp.int32),
            pltpu.VMEM((CH, 16), jnp.float32),
            pltpu.VMEM((64, 16), jnp.float32),
            pltpu.VMEM_SHARED((NROWS, 16), jnp.float32),
        ],
    )
    def k(col_hbm, out_hbm, idx_v, ones_v, z_v, acc_sh):
        c = lax.axis_index("c")
        s = lax.axis_index("s")
        wid = s * NC + c

        @pl.loop(0, 64)
        def _(i):
            z_v[i, :] = jnp.zeros((16,), jnp.float32)

        @pl.loop(0, CH)
        def _(i):
            ones_v[i, :] = jnp.ones((16,), jnp.float32)

        @pl.loop(0, RPT // 64)
        def _(kk):
            pltpu.sync_copy(z_v, acc_sh.at[pl.ds(s * RPT + kk * 64, 64)])

        pltpu.sync_copy(col_hbm.at[wid], idx_v)
        plsc.subcore_barrier()

        @pl.loop(0, CHA)
        def _(j):
            pltpu.sync_copy(ones_v, acc_sh.at[idx_v.at[j]], add=True)

        plsc.subcore_barrier()
        pltpu.sync_copy(acc_sh.at[pl.ds(s * RPT, RPT)],
                        out_hbm.at[c, pl.ds(s * RPT, RPT)])

    return k(col3a)


def _segment_sum(lo, hi, row3, col3):
    @functools.partial(
        pl.kernel,
        out_type=jax.ShapeDtypeStruct((NC, NROWS, H), jnp.float32),
        mesh=_sc_mesh(),
        scratch_types=[
            pltpu.VMEM((CHA, CH), jnp.int32),
            pltpu.VMEM((CHA, CH), jnp.int32),
            pltpu.VMEM((CH, H), jnp.float32),
            pltpu.VMEM((CH, H), jnp.float32),
            pltpu.SemaphoreType.DMA,
            pltpu.SemaphoreType.DMA,
            pltpu.VMEM_SHARED((NROWS, H), jnp.float32),
        ],
    )
    def k(lo_hbm, hi_hbm, row_hbm, col_hbm, out_hbm,
          ri_v, ci_v, g0_v, g1_v, sem0, sem1, acc_sh):
        c = lax.axis_index("c")
        s = lax.axis_index("s")
        bufs = (g0_v, g1_v)
        sems = (sem0, sem1)

        def gather_desc(jj, b):
            if_lo = pltpu.make_async_copy(
                lo_hbm.at[ri_v.at[jj]], bufs[b], sems[b])
            if_hi = pltpu.make_async_copy(
                hi_hbm.at[ri_v.at[jj]], bufs[b], sems[b])
            return if_lo, if_hi

        def start_gather(jj, b):
            if_lo, if_hi = gather_desc(jj, b)

            @pl.when(c == 0)
            def _():
                if_lo.start()

            @pl.when(c == 1)
            def _():
                if_hi.start()

        def wait_gather(jj, b):
            if_lo, if_hi = gather_desc(jj, b)

            @pl.when(c == 0)
            def _():
                if_lo.wait()

            @pl.when(c == 1)
            def _():
                if_hi.wait()

        @pl.loop(0, CH)
        def _(i):
            @pl.loop(0, H // 16)
            def _(j):
                g0_v[i, pl.ds(j * 16, 16)] = jnp.zeros((16,), jnp.float32)

        @pl.loop(0, RPT // CH)
        def _(kk):
            pltpu.sync_copy(g0_v, acc_sh.at[pl.ds(s * RPT + kk * CH, CH)])

        plsc.subcore_barrier()

        for h_stage in range(2):
            slab = h_stage * NS + s
            pltpu.sync_copy(row_hbm.at[slab], ri_v)
            pltpu.sync_copy(col_hbm.at[slab], ci_v)
            start_gather(0, 0)

            @pl.loop(0, CHA, step=2)
            def _(j):
                for b in range(2):
                    jj = j + b
                    wait_gather(jj, b)

                    @pl.when(jj + 1 < CHA)
                    def _():
                        start_gather(jj + 1, 1 - b)

                    pltpu.sync_copy(bufs[b], acc_sh.at[ci_v.at[jj]],
                                    add=True)

        plsc.subcore_barrier()
        pltpu.sync_copy(acc_sh.at[pl.ds(s * RPT, RPT)],
                        out_hbm.at[c, pl.ds(s * RPT, RPT)])

    return k(lo, hi, row3, col3)


def _scale_body(x_ref, w_ref, d_ref, lo_ref, hi_ref):
    xw = jnp.dot(x_ref[...], w_ref[...], preferred_element_type=jnp.float32)
    deg = d_ref[0, :, 0] + d_ref[1, :, 0] + 1.0
    dinv = lax.rsqrt(deg)[:, None]
    lo_ref[...] = xw[:, :H] * dinv
    hi_ref[...] = xw[:, H:] * dinv


def _scale(x, w, deg2):
    nb = 10
    bs = N // nb
    return pl.pallas_call(
        _scale_body,
        grid=(nb,),
        in_specs=[
            pl.BlockSpec((bs, D), lambda i: (i, 0)),
            pl.BlockSpec((D, D), lambda i: (0, 0)),
            pl.BlockSpec((NC, bs, 16), lambda i: (0, i, 0)),
        ],
        out_specs=[
            pl.BlockSpec((bs, H), lambda i: (i, 0)),
            pl.BlockSpec((bs, H), lambda i: (i, 0)),
        ],
        out_shape=[
            jax.ShapeDtypeStruct((N, H), jnp.float32),
            jax.ShapeDtypeStruct((N, H), jnp.float32),
        ],
    )(x, w, deg2)


def _fuse_body(hraw_ref, lo_ref, hi_ref, d_ref, bg_ref, t_ref, wt_ref,
               bt_ref, h_ref, st_ref, temb_s, acc_s):
    i = pl.program_id(0)

    @pl.when(i == 0)
    def _():
        temb_s[...] = jnp.maximum(
            jnp.dot(t_ref[...], wt_ref[...],
                    preferred_element_type=jnp.float32) + bt_ref[...], 0.0)
        acc_s[...] = jnp.zeros_like(acc_s)

    temb = temb_s[...]
    deg = d_ref[0, :, 0] + d_ref[1, :, 0] + 1.0
    dinv = lax.rsqrt(deg)[:, None]
    h_lo = jnp.maximum(
        dinv * (hraw_ref[0] + lo_ref[...]) + bg_ref[:, :H] + temb[:, :H], 0.0)
    h_hi = jnp.maximum(
        dinv * (hraw_ref[1] + hi_ref[...]) + bg_ref[:, H:] + temb[:, H:], 0.0)
    h = jnp.concatenate([h_lo, h_hi], axis=1)
    h_ref[...] = h
    acc_s[0:1, :] += jnp.sum(h, axis=0, keepdims=True)
    acc_s[1:2, :] += jnp.sum(h * h, axis=0, keepdims=True)
    st_ref[...] = acc_s[...]


def _fuse(hraw, lo, hi, deg2, bg, t, wt, bt):
    nb = 10
    bs = N // nb
    return pl.pallas_call(
        _fuse_body,
        grid=(nb,),
        in_specs=[
            pl.BlockSpec((NC, bs, H), lambda i: (0, i, 0)),
            pl.BlockSpec((bs, H), lambda i: (i, 0)),
            pl.BlockSpec((bs, H), lambda i: (i, 0)),
            pl.BlockSpec((NC, bs, 16), lambda i: (0, i, 0)),
            pl.BlockSpec((1, D), lambda i: (0, 0)),
            pl.BlockSpec((1, D), lambda i: (0, 0)),
            pl.BlockSpec((D, D), lambda i: (0, 0)),
            pl.BlockSpec((1, D), lambda i: (0, 0)),
        ],
        out_specs=[
            pl.BlockSpec((bs, D), lambda i: (i, 0)),
            pl.BlockSpec((8, D), lambda i: (0, 0)),
        ],
        out_shape=[
            jax.ShapeDtypeStruct((N, D), jnp.float32),
            jax.ShapeDtypeStruct((8, D), jnp.float32),
        ],
        scratch_shapes=[
            pltpu.VMEM((1, D), jnp.float32),
            pltpu.VMEM((8, D), jnp.float32),
        ],
    )(hraw, lo, hi, deg2, bg, t, wt, bt)


def _bn_body(h_ref, st_ref, g_ref, b_ref, o_ref):
    mean = st_ref[0:1, :] * (1.0 / N)
    var = st_ref[1:2, :] * (1.0 / N) - mean * mean
    scale = lax.rsqrt(var + 1e-5) * g_ref[...]
    o_ref[...] = (h_ref[...] - mean) * scale + b_ref[...]


def _bn(h, st, g, b):
    nb = 10
    bs = N // nb
    return pl.pallas_call(
        _bn_body,
        grid=(nb,),
        in_specs=[
            pl.BlockSpec((bs, D), lambda i: (i, 0)),
            pl.BlockSpec((8, D), lambda i: (0, 0)),
            pl.BlockSpec((1, D), lambda i: (0, 0)),
            pl.BlockSpec((1, D), lambda i: (0, 0)),
        ],
        out_specs=pl.BlockSpec((bs, D), lambda i: (i, 0)),
        out_shape=jax.ShapeDtypeStruct((N, D), jnp.float32),
    )(h, st, g, b)


def kernel(x, edge_index, t, W_gcn, b_gcn, W_t, b_t, gamma, beta):
    row = edge_index[0]
    col = edge_index[1]
    pad = EPAD - E
    rowp = jnp.concatenate([row, jnp.zeros((pad,), jnp.int32)])
    colp = jnp.concatenate([col, jnp.full((pad,), DUMMY, jnp.int32)])
    col3a = colp.reshape(NC * NS, CHA, CH)
    row3a = rowp.reshape(NC * NS, CHA, CH)

    deg2 = _degree(col3a)
    lo, hi = _scale(x, W_gcn, deg2)
    hraw = _segment_sum(lo, hi, row3a, col3a)
    h, st = _fuse(hraw, lo, hi, deg2, b_gcn.reshape(1, D), t, W_t,
                  b_t.reshape(1, D))
    return _bn(h, st, gamma.reshape(1, D), beta.reshape(1, D))


# speedup vs baseline: 10.2689x; 1.0430x over previous
import functools

import jax
import jax.numpy as jnp
from jax import lax
from jax.experimental import pallas as pl
from jax.experimental.pallas import tpu as pltpu
from jax.experimental.pallas import tpu_sc as plsc

N = 10000
E = 160000
D = 256
H = 128
NC = 2
NS = 16
CH = 128
EPAD = 163840
NROWS = 10240
DUMMY = 10000
RPT = NROWS // NS
CHA = EPAD // (NC * NS * CH)
CHC = EPAD // (NS * CH)


def _sc_mesh():
    return plsc.VectorSubcoreMesh(
        core_axis_name="c", subcore_axis_name="s", num_cores=NC,
        num_subcores=NS)


def _degree(col3a):
    @functools.partial(
        pl.kernel,
        out_type=jax.ShapeDtypeStruct((NC, NROWS, 16), jnp.float32),
        mesh=_sc_mesh(),
        scratch_types=[
            pltpu.VMEM((CHA, CH), jnp.int32),
            pltpu.VMEM((CH, 16), jnp.float32),
            pltpu.VMEM((64, 16), jnp.float32),
            pltpu.VMEM_SHARED((NROWS, 16), jnp.float32),
        ],
    )
    def k(col_hbm, out_hbm, idx_v, ones_v, z_v, acc_sh):
        c = lax.axis_index("c")
        s = lax.axis_index("s")
        wid = s * NC + c

        @pl.loop(0, 64)
        def _(i):
            z_v[i, :] = jnp.zeros((16,), jnp.float32)

        @pl.loop(0, CH)
        def _(i):
            ones_v[i, :] = jnp.ones((16,), jnp.float32)

        @pl.loop(0, RPT // 64)
        def _(kk):
            pltpu.sync_copy(z_v, acc_sh.at[pl.ds(s * RPT + kk * 64, 64)])

        pltpu.sync_copy(col_hbm.at[wid], idx_v)
        plsc.subcore_barrier()

        @pl.loop(0, CHA)
        def _(j):
            pltpu.sync_copy(ones_v, acc_sh.at[idx_v.at[j]], add=True)

        plsc.subcore_barrier()
        pltpu.sync_copy(acc_sh.at[pl.ds(s * RPT, RPT)],
                        out_hbm.at[c, pl.ds(s * RPT, RPT)])

    return k(col3a)


def _segment_sum(lo, hi, row3, col3):
    @functools.partial(
        pl.kernel,
        out_type=jax.ShapeDtypeStruct((NC, NROWS, H), jnp.float32),
        mesh=_sc_mesh(),
        scratch_types=[
            pltpu.VMEM((CHA, CH), jnp.int32),
            pltpu.VMEM((CHA, CH), jnp.int32),
            pltpu.VMEM((CH, H), jnp.float32),
            pltpu.VMEM((CH, H), jnp.float32),
            pltpu.SemaphoreType.DMA,
            pltpu.SemaphoreType.DMA,
            pltpu.VMEM_SHARED((NROWS, H), jnp.float32),
        ],
    )
    def k(lo_hbm, hi_hbm, row_hbm, col_hbm, out_hbm,
          ri_v, ci_v, g0_v, g1_v, sem0, sem1, acc_sh):
        c = lax.axis_index("c")
        s = lax.axis_index("s")
        bufs = (g0_v, g1_v)
        sems = (sem0, sem1)

        def gather_desc(jj, b):
            if_lo = pltpu.make_async_copy(
                lo_hbm.at[ri_v.at[jj]], bufs[b], sems[b])
            if_hi = pltpu.make_async_copy(
                hi_hbm.at[ri_v.at[jj]], bufs[b], sems[b])
            return if_lo, if_hi

        def start_gather(jj, b):
            if_lo, if_hi = gather_desc(jj, b)

            @pl.when(c == 0)
            def _():
                if_lo.start()

            @pl.when(c == 1)
            def _():
                if_hi.start()

        def wait_gather(jj, b):
            if_lo, if_hi = gather_desc(jj, b)

            @pl.when(c == 0)
            def _():
                if_lo.wait()

            @pl.when(c == 1)
            def _():
                if_hi.wait()

        @pl.loop(0, CH)
        def _(i):
            @pl.loop(0, H // 16)
            def _(j):
                g0_v[i, pl.ds(j * 16, 16)] = jnp.zeros((16,), jnp.float32)

        @pl.loop(0, RPT // CH)
        def _(kk):
            pltpu.sync_copy(g0_v, acc_sh.at[pl.ds(s * RPT + kk * CH, CH)])

        plsc.subcore_barrier()

        for h_stage in range(2):
            slab = h_stage * NS + s
            pltpu.sync_copy(row_hbm.at[slab], ri_v)
            pltpu.sync_copy(col_hbm.at[slab], ci_v)
            start_gather(0, 0)

            @pl.loop(0, CHA, step=2)
            def _(j):
                for b in range(2):
                    jj = j + b
                    wait_gather(jj, b)

                    @pl.when(jj + 1 < CHA)
                    def _():
                        start_gather(jj + 1, 1 - b)

                    pltpu.sync_copy(bufs[b], acc_sh.at[ci_v.at[jj]],
                                    add=True)

        plsc.subcore_barrier()
        pltpu.sync_copy(acc_sh.at[pl.ds(s * RPT, RPT)],
                        out_hbm.at[c, pl.ds(s * RPT, RPT)])

    return k(lo, hi, row3, col3)


def _mm_body(x_ref, w_ref, xw_ref):
    xw_ref[...] = jnp.dot(x_ref[...], w_ref[...],
                          preferred_element_type=jnp.float32)


def _matmul(x, w):
    nb = 10
    bs = N // nb
    return pl.pallas_call(
        _mm_body,
        grid=(nb,),
        in_specs=[
            pl.BlockSpec((bs, D), lambda i: (i, 0)),
            pl.BlockSpec((D, D), lambda i: (0, 0)),
        ],
        out_specs=pl.BlockSpec((bs, D), lambda i: (i, 0)),
        out_shape=jax.ShapeDtypeStruct((N, D), jnp.float32),
    )(x, w)


def _scale_body(xw_ref, d_ref, lo_ref, hi_ref):
    deg = d_ref[0, :, 0] + d_ref[1, :, 0] + 1.0
    dinv = lax.rsqrt(deg)[:, None]
    xw = xw_ref[...]
    lo_ref[...] = xw[:, :H] * dinv
    hi_ref[...] = xw[:, H:] * dinv


def _scale(xw, deg2):
    nb = 10
    bs = N // nb
    return pl.pallas_call(
        _scale_body,
        grid=(nb,),
        in_specs=[
            pl.BlockSpec((bs, D), lambda i: (i, 0)),
            pl.BlockSpec((NC, bs, 16), lambda i: (0, i, 0)),
        ],
        out_specs=[
            pl.BlockSpec((bs, H), lambda i: (i, 0)),
            pl.BlockSpec((bs, H), lambda i: (i, 0)),
        ],
        out_shape=[
            jax.ShapeDtypeStruct((N, H), jnp.float32),
            jax.ShapeDtypeStruct((N, H), jnp.float32),
        ],
    )(xw, deg2)


def _fuse_body(hraw_ref, lo_ref, hi_ref, d_ref, bg_ref, t_ref, wt_ref,
               bt_ref, h_ref, st_ref, temb_s, acc_s):
    i = pl.program_id(0)

    @pl.when(i == 0)
    def _():
        temb_s[...] = jnp.maximum(
            jnp.dot(t_ref[...], wt_ref[...],
                    preferred_element_type=jnp.float32) + bt_ref[...], 0.0)
        acc_s[...] = jnp.zeros_like(acc_s)

    temb = temb_s[...]
    deg = d_ref[0, :, 0] + d_ref[1, :, 0] + 1.0
    dinv = lax.rsqrt(deg)[:, None]
    h_lo = jnp.maximum(
        dinv * (hraw_ref[0] + lo_ref[...]) + bg_ref[:, :H] + temb[:, :H], 0.0)
    h_hi = jnp.maximum(
        dinv * (hraw_ref[1] + hi_ref[...]) + bg_ref[:, H:] + temb[:, H:], 0.0)
    h = jnp.concatenate([h_lo, h_hi], axis=1)
    h_ref[...] = h
    acc_s[0:1, :] += jnp.sum(h, axis=0, keepdims=True)
    acc_s[1:2, :] += jnp.sum(h * h, axis=0, keepdims=True)
    st_ref[...] = acc_s[...]


def _fuse(hraw, lo, hi, deg2, bg, t, wt, bt):
    nb = 10
    bs = N // nb
    return pl.pallas_call(
        _fuse_body,
        grid=(nb,),
        in_specs=[
            pl.BlockSpec((NC, bs, H), lambda i: (0, i, 0)),
            pl.BlockSpec((bs, H), lambda i: (i, 0)),
            pl.BlockSpec((bs, H), lambda i: (i, 0)),
            pl.BlockSpec((NC, bs, 16), lambda i: (0, i, 0)),
            pl.BlockSpec((1, D), lambda i: (0, 0)),
            pl.BlockSpec((1, D), lambda i: (0, 0)),
            pl.BlockSpec((D, D), lambda i: (0, 0)),
            pl.BlockSpec((1, D), lambda i: (0, 0)),
        ],
        out_specs=[
            pl.BlockSpec((bs, D), lambda i: (i, 0)),
            pl.BlockSpec((8, D), lambda i: (0, 0)),
        ],
        out_shape=[
            jax.ShapeDtypeStruct((N, D), jnp.float32),
            jax.ShapeDtypeStruct((8, D), jnp.float32),
        ],
        scratch_shapes=[
            pltpu.VMEM((1, D), jnp.float32),
            pltpu.VMEM((8, D), jnp.float32),
        ],
    )(hraw, lo, hi, deg2, bg, t, wt, bt)


def _bn_body(h_ref, st_ref, g_ref, b_ref, o_ref):
    mean = st_ref[0:1, :] * (1.0 / N)
    var = st_ref[1:2, :] * (1.0 / N) - mean * mean
    scale = lax.rsqrt(var + 1e-5) * g_ref[...]
    o_ref[...] = (h_ref[...] - mean) * scale + b_ref[...]


def _bn(h, st, g, b):
    nb = 10
    bs = N // nb
    return pl.pallas_call(
        _bn_body,
        grid=(nb,),
        in_specs=[
            pl.BlockSpec((bs, D), lambda i: (i, 0)),
            pl.BlockSpec((8, D), lambda i: (0, 0)),
            pl.BlockSpec((1, D), lambda i: (0, 0)),
            pl.BlockSpec((1, D), lambda i: (0, 0)),
        ],
        out_specs=pl.BlockSpec((bs, D), lambda i: (i, 0)),
        out_shape=jax.ShapeDtypeStruct((N, D), jnp.float32),
    )(h, st, g, b)


def kernel(x, edge_index, t, W_gcn, b_gcn, W_t, b_t, gamma, beta):
    row = edge_index[0]
    col = edge_index[1]
    pad = EPAD - E
    rowp = jnp.concatenate([row, jnp.zeros((pad,), jnp.int32)])
    colp = jnp.concatenate([col, jnp.full((pad,), DUMMY, jnp.int32)])
    col3a = colp.reshape(NC * NS, CHA, CH)
    row3a = rowp.reshape(NC * NS, CHA, CH)

    deg2 = _degree(col3a)
    xw = _matmul(x, W_gcn)
    lo, hi = _scale(xw, deg2)
    hraw = _segment_sum(lo, hi, row3a, col3a)
    h, st = _fuse(hraw, lo, hi, deg2, b_gcn.reshape(1, D), t, W_t,
                  b_t.reshape(1, D))
    return _bn(h, st, gamma.reshape(1, D), beta.reshape(1, D))
